# baseline (speedup 1.0000x reference)
import functools

import jax
import jax.numpy as jnp
from jax import lax
from jax.experimental import pallas as pl
from jax.experimental.pallas import tpu as pltpu
from jax.experimental.pallas import tpu_sc as plsc

BATCH = 16384
DIM = 64
NROWS = 1000000
_FULL = (NROWS // 128) * 128
_NBLK = _FULL // 128

_NC = 2
_NS = 16
_NW = _NC * _NS
_BPW = BATCH // _NW
_CHUNK = 128


def _transpose_block(in_b, out_b, iota16):
    def per_row(r, _):
        rv = jnp.full((16,), r, jnp.int32)
        for j0 in range(0, DIM, 16):
            vals = plsc.load_gather(in_b, [iota16 + j0, rv])
            plsc.store_scatter(out_b, [rv, iota16 + j0], vals)
        return 0

    lax.fori_loop(0, 128, per_row, 0)


def _sc_relayout_body(gu_t, gi_t, out_gu, out_gi,
                      in_a, in_b, out_a, out_b, sem_ia, sem_ib,
                      sem_oa, sem_ob):
    wid = lax.axis_index("s") * _NC + lax.axis_index("c")
    iota16 = lax.iota(jnp.int32, 16)
    tabs = (gu_t, gi_t)
    outs = (out_gu, out_gi)
    ibufs = (in_a, in_b)
    obufs = (out_a, out_b)
    isems = (sem_ia, sem_ib)
    osems = (sem_oa, sem_ob)

    npert = (_NBLK + _NW - 1) // _NW

    def fetch(slot, t, blk):
        col = pl.multiple_of(blk * 128, 128)
        pltpu.make_async_copy(
            tabs[t].at[:, pl.ds(col, 128)], ibufs[t], isems[t]).start()

    def blk_of(i):
        return wid + i * _NW

    @pl.when(blk_of(0) < _NBLK)
    def _():
        fetch(0, 0, blk_of(0))
        fetch(0, 1, blk_of(0))

    def step(i, _):
        blk = blk_of(i)

        @pl.when(blk < _NBLK)
        def _():
            for t in range(2):
                pltpu.make_async_copy(
                    tabs[t].at[:, pl.ds(0, 128)], ibufs[t], isems[t]).wait()
                _transpose_block(ibufs[t], obufs[t], iota16)

            @pl.when(blk_of(i + 1) < _NBLK)
            def _():
                nxt = pl.multiple_of(blk_of(i + 1) * 128, 128)
                for t in range(2):
                    pltpu.make_async_copy(
                        tabs[t].at[:, pl.ds(nxt, 128)], ibufs[t],
                        isems[t]).start()

            for t in range(2):
                pltpu.make_async_copy(
                    obufs[t], outs[t].at[pl.ds(blk * 128, 128)],
                    osems[t]).start()
            for t in range(2):
                pltpu.make_async_copy(
                    obufs[t], outs[t].at[pl.ds(blk * 128, 128)],
                    osems[t]).wait()
        return 0

    lax.fori_loop(0, npert, step, 0)


@functools.cache
def _make_sc_relayout():
    return functools.partial(
        pl.kernel,
        out_type=[jax.ShapeDtypeStruct((NROWS, DIM), jnp.float32)] * 2,
        mesh=plsc.VectorSubcoreMesh(core_axis_name="c", subcore_axis_name="s"),
        compiler_params=pltpu.CompilerParams(needs_layout_passes=False),
        scratch_types=[
            pltpu.VMEM((DIM, 128), jnp.float32),
            pltpu.VMEM((DIM, 128), jnp.float32),
            pltpu.VMEM((128, DIM), jnp.float32),
            pltpu.VMEM((128, DIM), jnp.float32),
        ] + [pltpu.SemaphoreType.DMA] * 4,
    )(_sc_relayout_body)


def _sc_gather_body(uc, ic, uids, iids, gu_t, gi_t, mu_t, mi_t,
                    out_gu, out_gi, out_mu, out_mi,
                    idx_vm, bufs, sems):
    wid = lax.axis_index("s") * _NC + lax.axis_index("c")
    base = wid * _BPW
    srcs = (uc, ic, uids, iids)
    for t in range(4):
        pltpu.sync_copy(srcs[t].at[pl.ds(base, _BPW)],
                        idx_vm.at[t])
    tables = (gu_t, gi_t, mu_t, mi_t)
    outs = (out_gu, out_gi, out_mu, out_mi)

    for c in range(_BPW // _CHUNK):
        def issue(g, _):
            vecs = [idx_vm[t, pl.ds(c * _CHUNK + g * 16, 16)]
                    for t in range(4)]
            for j in range(16):
                k = g * 16 + j
                for t in range(4):
                    pltpu.make_async_copy(
                        tables[t].at[pl.ds(vecs[t][j], 1)],
                        bufs[t].at[pl.ds(k, 1)], sems[t]).start()
            return 0

        lax.fori_loop(0, _CHUNK // 16, issue, 0)
        for t in range(4):
            pltpu.make_async_copy(
                tables[t].at[pl.ds(0, _CHUNK)], bufs[t], sems[t]).wait()
            pltpu.sync_copy(
                bufs[t], outs[t].at[pl.ds(base + c * _CHUNK, _CHUNK)])


@functools.cache
def _make_sc_gather():
    def body(uc, ic, uids, iids, gu_t, gi_t, mu_t, mi_t,
             out_gu, out_gi, out_mu, out_mi,
             idx_vm, b0, b1, b2, b3, s0, s1, s2, s3):
        _sc_gather_body(uc, ic, uids, iids, gu_t, gi_t, mu_t, mi_t,
                        out_gu, out_gi, out_mu, out_mi,
                        idx_vm, (b0, b1, b2, b3), (s0, s1, s2, s3))

    return functools.partial(
        pl.kernel,
        out_type=[jax.ShapeDtypeStruct((BATCH, DIM), jnp.float32)] * 4,
        mesh=plsc.VectorSubcoreMesh(core_axis_name="c", subcore_axis_name="s"),
        scratch_types=[
            pltpu.VMEM((4, _BPW), jnp.int32),
        ] + [pltpu.VMEM((_CHUNK, DIM), jnp.float32)] * 4
          + [pltpu.SemaphoreType.DMA] * 4,
    )(body)


_BLK = 2048


def _tc_mlp_body(gu, gi, mu, mi, uid, iid, tgu, tgi,
                 w0u, w0i, b0, s0, t0,
                 w1, b1, s1, t1,
                 w2, b2, s2, t2,
                 wg, wx, bo, out):
    iota = lax.broadcasted_iota(jnp.int32, (1, DIM), 1)
    du = uid[...] - _FULL
    di = iid[...] - _FULL
    oh_u = (du == iota).astype(jnp.float32)
    oh_i = (di == iota).astype(jnp.float32)
    gu_x = jnp.where(du >= 0, oh_u @ tgu[...], gu[...])
    gi_x = jnp.where(di >= 0, oh_i @ tgi[...], gi[...])

    x = mu[...] @ w0u[...] + mi[...] @ w0i[...] + b0[...]
    x = jnp.maximum(x, 0.0) * s0[...] + t0[...]
    x = x @ w1[...] + b1[...]
    x = jnp.maximum(x, 0.0) * s1[...] + t1[...]
    x = x @ w2[...] + b2[...]
    x = jnp.maximum(x, 0.0) * s2[...] + t2[...]
    g = gu_x * gi_x
    logit = (jnp.sum(g * wg[...], axis=1, keepdims=True)
             + jnp.sum(x * wx[...], axis=1, keepdims=True) + bo[...])
    out[...] = jax.nn.sigmoid(logit)


def _tc_mlp(gu, gi, mu, mi, uid2, iid2, params):
    n_blk = BATCH // _BLK
    data_spec = pl.BlockSpec((_BLK, DIM), lambda i: (i, 0))
    id_spec = pl.BlockSpec((_BLK, 1), lambda i: (i, 0))

    def full(a):
        return pl.BlockSpec(a.shape, lambda i: (0,) * a.ndim)

    in_specs = ([data_spec] * 4 + [id_spec] * 2
                + [full(p) for p in params])
    return pl.pallas_call(
        _tc_mlp_body,
        grid=(n_blk,),
        in_specs=in_specs,
        out_specs=pl.BlockSpec((_BLK, 1), lambda i: (i, 0)),
        out_shape=jax.ShapeDtypeStruct((BATCH, 1), jnp.float32),
    )(gu, gi, mu, mi, uid2, iid2, *params)


def kernel(inputs, gmf_user_table, gmf_item_table, mlp_user_table, mlp_item_table,
           W0, b0, g0, be0, m0, v0,
           W1, b1, g1, be1, m1, v1,
           W2, b2, g2, be2, m2, v2,
           Wout, bout):
    uids = inputs[:, 0].astype(jnp.int32)
    iids = inputs[:, 1].astype(jnp.int32)
    uc = jnp.minimum(uids, _FULL - 1)
    ic = jnp.minimum(iids, _FULL - 1)

    rgu, rgi = _make_sc_relayout()(gmf_user_table.T, gmf_item_table.T)

    gu, gi, mu, mi = _make_sc_gather()(
        uc, ic, uids, iids, rgu, rgi, mlp_user_table, mlp_item_table)

    def fold(g, be, m, v):
        s = g / jnp.sqrt(v + 1e-3)
        return s, be - m * s

    s0, t0 = fold(g0, be0, m0, v0)
    s1, t1 = fold(g1, be1, m1, v1)
    s2, t2 = fold(g2, be2, m2, v2)

    def row(a):
        return a.reshape(1, -1)

    tgu = gmf_user_table[_FULL:]
    tgi = gmf_item_table[_FULL:]

    params = [
        tgu, tgi,
        W0[:DIM], W0[DIM:], row(b0), row(s0), row(t0),
        W1, row(b1), row(s1), row(t1),
        W2, row(b2), row(s2), row(t2),
        row(Wout[:DIM, 0]), row(Wout[DIM:, 0]), row(bout),
    ]
    out = _tc_mlp(gu, gi, mu, mi,
                  uids.reshape(-1, 1), iids.reshape(-1, 1), params)
    return jnp.squeeze(out, axis=1)


# speedup vs baseline: 1.1989x; 1.1989x over previous
import functools

import jax
import jax.numpy as jnp
from jax import lax
from jax.experimental import pallas as pl
from jax.experimental.pallas import tpu as pltpu
from jax.experimental.pallas import tpu_sc as plsc

BATCH = 16384
DIM = 64
NROWS = 1000000
_FULL = (NROWS // 128) * 128
_NBLK = _FULL // 128

_NC = 2
_NS = 16
_NW = _NC * _NS
_BPW = BATCH // _NW
_CHUNK = 128


def _transpose_block(in_b, out_b, iota16):
    iotas = [iota16 + g * 16 for g in range(8)]

    def per_j(j, _):
        jv = jnp.full((16,), j, jnp.int32)
        for g in range(8):
            vals = plsc.load_gather(in_b, [jv, iotas[g]])
            plsc.store_scatter(out_b, [iotas[g], jv], vals)
        return 0

    lax.fori_loop(0, DIM, per_j, 0, unroll=2)


def _sc_relayout_body(gu_t, gi_t, out_gu, out_gi,
                      in_a, in_b, out_a, out_b, sem_ia, sem_ib,
                      sem_oa, sem_ob):
    wid = lax.axis_index("s") * _NC + lax.axis_index("c")
    iota16 = lax.iota(jnp.int32, 16)
    tabs = (gu_t, gi_t)
    outs = (out_gu, out_gi)
    ibufs = (in_a, in_b)
    obufs = (out_a, out_b)
    isems = (sem_ia, sem_ib)
    osems = (sem_oa, sem_ob)

    npert = (_NBLK + _NW - 1) // _NW

    def fetch(slot, t, blk):
        col = pl.multiple_of(blk * 128, 128)
        pltpu.make_async_copy(
            tabs[t].at[:, pl.ds(col, 128)], ibufs[t], isems[t]).start()

    def blk_of(i):
        return wid + i * _NW

    @pl.when(blk_of(0) < _NBLK)
    def _():
        fetch(0, 0, blk_of(0))
        fetch(0, 1, blk_of(0))

    def step(i, _):
        blk = blk_of(i)

        @pl.when(blk < _NBLK)
        def _():
            for t in range(2):
                pltpu.make_async_copy(
                    tabs[t].at[:, pl.ds(0, 128)], ibufs[t], isems[t]).wait()
                _transpose_block(ibufs[t], obufs[t], iota16)

            @pl.when(blk_of(i + 1) < _NBLK)
            def _():
                nxt = pl.multiple_of(blk_of(i + 1) * 128, 128)
                for t in range(2):
                    pltpu.make_async_copy(
                        tabs[t].at[:, pl.ds(nxt, 128)], ibufs[t],
                        isems[t]).start()

            for t in range(2):
                pltpu.make_async_copy(
                    obufs[t], outs[t].at[pl.ds(blk * 128, 128)],
                    osems[t]).start()
            for t in range(2):
                pltpu.make_async_copy(
                    obufs[t], outs[t].at[pl.ds(blk * 128, 128)],
                    osems[t]).wait()
        return 0

    lax.fori_loop(0, npert, step, 0)


@functools.cache
def _make_sc_relayout():
    return functools.partial(
        pl.kernel,
        out_type=[jax.ShapeDtypeStruct((NROWS, DIM), jnp.float32)] * 2,
        mesh=plsc.VectorSubcoreMesh(core_axis_name="c", subcore_axis_name="s"),
        compiler_params=pltpu.CompilerParams(needs_layout_passes=False),
        scratch_types=[
            pltpu.VMEM((DIM, 128), jnp.float32),
            pltpu.VMEM((DIM, 128), jnp.float32),
            pltpu.VMEM((128, DIM), jnp.float32),
            pltpu.VMEM((128, DIM), jnp.float32),
        ] + [pltpu.SemaphoreType.DMA] * 4,
    )(_sc_relayout_body)


def _sc_gather_body(uc, ic, uids, iids, gu_t, gi_t, mu_t, mi_t,
                    out_gu, out_gi, out_mu, out_mi,
                    idx_vm, bufs, sems):
    wid = lax.axis_index("s") * _NC + lax.axis_index("c")
    base = wid * _BPW
    srcs = (uc, ic, uids, iids)
    for t in range(4):
        pltpu.sync_copy(srcs[t].at[pl.ds(base, _BPW)],
                        idx_vm.at[t])
    tables = (gu_t, gi_t, mu_t, mi_t)
    outs = (out_gu, out_gi, out_mu, out_mi)

    for c in range(_BPW // _CHUNK):
        def issue(g, _):
            vecs = [idx_vm[t, pl.ds(c * _CHUNK + g * 16, 16)]
                    for t in range(4)]
            for j in range(16):
                k = g * 16 + j
                for t in range(4):
                    pltpu.make_async_copy(
                        tables[t].at[pl.ds(vecs[t][j], 1)],
                        bufs[t].at[pl.ds(k, 1)], sems[t]).start()
            return 0

        lax.fori_loop(0, _CHUNK // 16, issue, 0)
        for t in range(4):
            pltpu.make_async_copy(
                tables[t].at[pl.ds(0, _CHUNK)], bufs[t], sems[t]).wait()
            pltpu.sync_copy(
                bufs[t], outs[t].at[pl.ds(base + c * _CHUNK, _CHUNK)])


@functools.cache
def _make_sc_gather():
    def body(uc, ic, uids, iids, gu_t, gi_t, mu_t, mi_t,
             out_gu, out_gi, out_mu, out_mi,
             idx_vm, b0, b1, b2, b3, s0, s1, s2, s3):
        _sc_gather_body(uc, ic, uids, iids, gu_t, gi_t, mu_t, mi_t,
                        out_gu, out_gi, out_mu, out_mi,
                        idx_vm, (b0, b1, b2, b3), (s0, s1, s2, s3))

    return functools.partial(
        pl.kernel,
        out_type=[jax.ShapeDtypeStruct((BATCH, DIM), jnp.float32)] * 4,
        mesh=plsc.VectorSubcoreMesh(core_axis_name="c", subcore_axis_name="s"),
        scratch_types=[
            pltpu.VMEM((4, _BPW), jnp.int32),
        ] + [pltpu.VMEM((_CHUNK, DIM), jnp.float32)] * 4
          + [pltpu.SemaphoreType.DMA] * 4,
    )(body)


_BLK = 2048


def _tc_mlp_body(gu, gi, mu, mi, uid, iid, tgu, tgi,
                 w0u, w0i, b0, s0, t0,
                 w1, b1, s1, t1,
                 w2, b2, s2, t2,
                 wg, wx, bo, out):
    iota = lax.broadcasted_iota(jnp.int32, (1, DIM), 1)
    du = uid[...] - _FULL
    di = iid[...] - _FULL
    oh_u = (du == iota).astype(jnp.float32)
    oh_i = (di == iota).astype(jnp.float32)
    gu_x = jnp.where(du >= 0, oh_u @ tgu[...], gu[...])
    gi_x = jnp.where(di >= 0, oh_i @ tgi[...], gi[...])

    x = mu[...] @ w0u[...] + mi[...] @ w0i[...] + b0[...]
    x = jnp.maximum(x, 0.0) * s0[...] + t0[...]
    x = x @ w1[...] + b1[...]
    x = jnp.maximum(x, 0.0) * s1[...] + t1[...]
    x = x @ w2[...] + b2[...]
    x = jnp.maximum(x, 0.0) * s2[...] + t2[...]
    g = gu_x * gi_x
    logit = (jnp.sum(g * wg[...], axis=1, keepdims=True)
             + jnp.sum(x * wx[...], axis=1, keepdims=True) + bo[...])
    out[...] = jax.nn.sigmoid(logit)


def _tc_mlp(gu, gi, mu, mi, uid2, iid2, params):
    n_blk = BATCH // _BLK
    data_spec = pl.BlockSpec((_BLK, DIM), lambda i: (i, 0))
    id_spec = pl.BlockSpec((_BLK, 1), lambda i: (i, 0))

    def full(a):
        return pl.BlockSpec(a.shape, lambda i: (0,) * a.ndim)

    in_specs = ([data_spec] * 4 + [id_spec] * 2
                + [full(p) for p in params])
    return pl.pallas_call(
        _tc_mlp_body,
        grid=(n_blk,),
        in_specs=in_specs,
        out_specs=pl.BlockSpec((_BLK, 1), lambda i: (i, 0)),
        out_shape=jax.ShapeDtypeStruct((BATCH, 1), jnp.float32),
    )(gu, gi, mu, mi, uid2, iid2, *params)


def kernel(inputs, gmf_user_table, gmf_item_table, mlp_user_table, mlp_item_table,
           W0, b0, g0, be0, m0, v0,
           W1, b1, g1, be1, m1, v1,
           W2, b2, g2, be2, m2, v2,
           Wout, bout):
    uids = inputs[:, 0].astype(jnp.int32)
    iids = inputs[:, 1].astype(jnp.int32)
    uc = jnp.minimum(uids, _FULL - 1)
    ic = jnp.minimum(iids, _FULL - 1)

    rgu, rgi = _make_sc_relayout()(gmf_user_table.T, gmf_item_table.T)

    gu, gi, mu, mi = _make_sc_gather()(
        uc, ic, uids, iids, rgu, rgi, mlp_user_table, mlp_item_table)

    def fold(g, be, m, v):
        s = g / jnp.sqrt(v + 1e-3)
        return s, be - m * s

    s0, t0 = fold(g0, be0, m0, v0)
    s1, t1 = fold(g1, be1, m1, v1)
    s2, t2 = fold(g2, be2, m2, v2)

    def row(a):
        return a.reshape(1, -1)

    tgu = gmf_user_table[_FULL:]
    tgi = gmf_item_table[_FULL:]

    params = [
        tgu, tgi,
        W0[:DIM], W0[DIM:], row(b0), row(s0), row(t0),
        W1, row(b1), row(s1), row(t1),
        W2, row(b2), row(s2), row(t2),
        row(Wout[:DIM, 0]), row(Wout[DIM:, 0]), row(bout),
    ]
    out = _tc_mlp(gu, gi, mu, mi,
                  uids.reshape(-1, 1), iids.reshape(-1, 1), params)
    return jnp.squeeze(out, axis=1)


# speedup vs baseline: 1.8369x; 1.5321x over previous
import functools

import jax
import jax.numpy as jnp
from jax import lax
from jax.experimental import pallas as pl
from jax.experimental.pallas import tpu as pltpu
from jax.experimental.pallas import tpu_sc as plsc

BATCH = 16384
DIM = 64
NROWS = 1000000
_FULL = (NROWS // 128) * 128
_NBLK = _FULL // 128

_NC = 2
_NS = 16
_NW = _NC * _NS
_BPW = BATCH // _NW
_CHUNK = 128


def _transpose_block(in_b, out_b, iota16):
    rolls = [(iota16 + c) % 16 for c in range(16)]

    def per_idblk(g, _):
        idv = iota16 + g * 16
        for j0 in range(0, DIM, 16):
            for c in range(16):
                jv = rolls[c] + j0
                vals = plsc.load_gather(in_b, [jv, idv])
                plsc.store_scatter(out_b, [idv, jv], vals)
        return 0

    lax.fori_loop(0, 8, per_idblk, 0)


def _sc_relayout_body(gu_t, gi_t, out_gu, out_gi,
                      in_a, in_b, out_a, out_b, sem_ia, sem_ib,
                      sem_oa, sem_ob):
    wid = lax.axis_index("s") * _NC + lax.axis_index("c")
    iota16 = lax.iota(jnp.int32, 16)
    tabs = (gu_t, gi_t)
    outs = (out_gu, out_gi)
    ibufs = (in_a, in_b)
    obufs = (out_a, out_b)
    isems = (sem_ia, sem_ib)
    osems = (sem_oa, sem_ob)

    npert = (_NBLK + _NW - 1) // _NW

    def fetch(slot, t, blk):
        col = pl.multiple_of(blk * 128, 128)
        pltpu.make_async_copy(
            tabs[t].at[:, pl.ds(col, 128)], ibufs[t], isems[t]).start()

    def blk_of(i):
        return wid + i * _NW

    @pl.when(blk_of(0) < _NBLK)
    def _():
        fetch(0, 0, blk_of(0))
        fetch(0, 1, blk_of(0))

    def step(i, _):
        blk = blk_of(i)

        @pl.when(blk < _NBLK)
        def _():
            for t in range(2):
                pltpu.make_async_copy(
                    tabs[t].at[:, pl.ds(0, 128)], ibufs[t], isems[t]).wait()
                _transpose_block(ibufs[t], obufs[t], iota16)

            @pl.when(blk_of(i + 1) < _NBLK)
            def _():
                nxt = pl.multiple_of(blk_of(i + 1) * 128, 128)
                for t in range(2):
                    pltpu.make_async_copy(
                        tabs[t].at[:, pl.ds(nxt, 128)], ibufs[t],
                        isems[t]).start()

            for t in range(2):
                pltpu.make_async_copy(
                    obufs[t], outs[t].at[pl.ds(blk * 128, 128)],
                    osems[t]).start()
            for t in range(2):
                pltpu.make_async_copy(
                    obufs[t], outs[t].at[pl.ds(blk * 128, 128)],
                    osems[t]).wait()
        return 0

    lax.fori_loop(0, npert, step, 0)


@functools.cache
def _make_sc_relayout():
    return functools.partial(
        pl.kernel,
        out_type=[jax.ShapeDtypeStruct((NROWS, DIM), jnp.float32)] * 2,
        mesh=plsc.VectorSubcoreMesh(core_axis_name="c", subcore_axis_name="s"),
        compiler_params=pltpu.CompilerParams(needs_layout_passes=False),
        scratch_types=[
            pltpu.VMEM((DIM, 128), jnp.float32),
            pltpu.VMEM((DIM, 128), jnp.float32),
            pltpu.VMEM((128, DIM), jnp.float32),
            pltpu.VMEM((128, DIM), jnp.float32),
        ] + [pltpu.SemaphoreType.DMA] * 4,
    )(_sc_relayout_body)


def _sc_gather_body(uc, ic, uids, iids, gu_t, gi_t, mu_t, mi_t,
                    out_gu, out_gi, out_mu, out_mi,
                    idx_vm, bufs, sems):
    wid = lax.axis_index("s") * _NC + lax.axis_index("c")
    base = wid * _BPW
    srcs = (uc, ic, uids, iids)
    for t in range(4):
        pltpu.sync_copy(srcs[t].at[pl.ds(base, _BPW)],
                        idx_vm.at[t])
    tables = (gu_t, gi_t, mu_t, mi_t)
    outs = (out_gu, out_gi, out_mu, out_mi)

    for c in range(_BPW // _CHUNK):
        def issue(g, _):
            vecs = [idx_vm[t, pl.ds(c * _CHUNK + g * 16, 16)]
                    for t in range(4)]
            for j in range(16):
                k = g * 16 + j
                for t in range(4):
                    pltpu.make_async_copy(
                        tables[t].at[pl.ds(vecs[t][j], 1)],
                        bufs[t].at[pl.ds(k, 1)], sems[t]).start()
            return 0

        lax.fori_loop(0, _CHUNK // 16, issue, 0)
        for t in range(4):
            pltpu.make_async_copy(
                tables[t].at[pl.ds(0, _CHUNK)], bufs[t], sems[t]).wait()
            pltpu.sync_copy(
                bufs[t], outs[t].at[pl.ds(base + c * _CHUNK, _CHUNK)])


@functools.cache
def _make_sc_gather():
    def body(uc, ic, uids, iids, gu_t, gi_t, mu_t, mi_t,
             out_gu, out_gi, out_mu, out_mi,
             idx_vm, b0, b1, b2, b3, s0, s1, s2, s3):
        _sc_gather_body(uc, ic, uids, iids, gu_t, gi_t, mu_t, mi_t,
                        out_gu, out_gi, out_mu, out_mi,
                        idx_vm, (b0, b1, b2, b3), (s0, s1, s2, s3))

    return functools.partial(
        pl.kernel,
        out_type=[jax.ShapeDtypeStruct((BATCH, DIM), jnp.float32)] * 4,
        mesh=plsc.VectorSubcoreMesh(core_axis_name="c", subcore_axis_name="s"),
        scratch_types=[
            pltpu.VMEM((4, _BPW), jnp.int32),
        ] + [pltpu.VMEM((_CHUNK, DIM), jnp.float32)] * 4
          + [pltpu.SemaphoreType.DMA] * 4,
    )(body)


_BLK = 2048


def _tc_mlp_body(gu, gi, mu, mi, uid, iid, tgu, tgi,
                 w0u, w0i, b0, s0, t0,
                 w1, b1, s1, t1,
                 w2, b2, s2, t2,
                 wg, wx, bo, out):
    iota = lax.broadcasted_iota(jnp.int32, (1, DIM), 1)
    du = uid[...] - _FULL
    di = iid[...] - _FULL
    oh_u = (du == iota).astype(jnp.float32)
    oh_i = (di == iota).astype(jnp.float32)
    gu_x = jnp.where(du >= 0, oh_u @ tgu[...], gu[...])
    gi_x = jnp.where(di >= 0, oh_i @ tgi[...], gi[...])

    x = mu[...] @ w0u[...] + mi[...] @ w0i[...] + b0[...]
    x = jnp.maximum(x, 0.0) * s0[...] + t0[...]
    x = x @ w1[...] + b1[...]
    x = jnp.maximum(x, 0.0) * s1[...] + t1[...]
    x = x @ w2[...] + b2[...]
    x = jnp.maximum(x, 0.0) * s2[...] + t2[...]
    g = gu_x * gi_x
    logit = (jnp.sum(g * wg[...], axis=1, keepdims=True)
             + jnp.sum(x * wx[...], axis=1, keepdims=True) + bo[...])
    out[...] = jax.nn.sigmoid(logit)


def _tc_mlp(gu, gi, mu, mi, uid2, iid2, params):
    n_blk = BATCH // _BLK
    data_spec = pl.BlockSpec((_BLK, DIM), lambda i: (i, 0))
    id_spec = pl.BlockSpec((_BLK, 1), lambda i: (i, 0))

    def full(a):
        return pl.BlockSpec(a.shape, lambda i: (0,) * a.ndim)

    in_specs = ([data_spec] * 4 + [id_spec] * 2
                + [full(p) for p in params])
    return pl.pallas_call(
        _tc_mlp_body,
        grid=(n_blk,),
        in_specs=in_specs,
        out_specs=pl.BlockSpec((_BLK, 1), lambda i: (i, 0)),
        out_shape=jax.ShapeDtypeStruct((BATCH, 1), jnp.float32),
    )(gu, gi, mu, mi, uid2, iid2, *params)


def kernel(inputs, gmf_user_table, gmf_item_table, mlp_user_table, mlp_item_table,
           W0, b0, g0, be0, m0, v0,
           W1, b1, g1, be1, m1, v1,
           W2, b2, g2, be2, m2, v2,
           Wout, bout):
    uids = inputs[:, 0].astype(jnp.int32)
    iids = inputs[:, 1].astype(jnp.int32)
    uc = jnp.minimum(uids, _FULL - 1)
    ic = jnp.minimum(iids, _FULL - 1)

    rgu, rgi = _make_sc_relayout()(gmf_user_table.T, gmf_item_table.T)

    gu, gi, mu, mi = _make_sc_gather()(
        uc, ic, uids, iids, rgu, rgi, mlp_user_table, mlp_item_table)

    def fold(g, be, m, v):
        s = g / jnp.sqrt(v + 1e-3)
        return s, be - m * s

    s0, t0 = fold(g0, be0, m0, v0)
    s1, t1 = fold(g1, be1, m1, v1)
    s2, t2 = fold(g2, be2, m2, v2)

    def row(a):
        return a.reshape(1, -1)

    tgu = gmf_user_table[_FULL:]
    tgi = gmf_item_table[_FULL:]

    params = [
        tgu, tgi,
        W0[:DIM], W0[DIM:], row(b0), row(s0), row(t0),
        W1, row(b1), row(s1), row(t1),
        W2, row(b2), row(s2), row(t2),
        row(Wout[:DIM, 0]), row(Wout[DIM:, 0]), row(bout),
    ]
    out = _tc_mlp(gu, gi, mu, mi,
                  uids.reshape(-1, 1), iids.reshape(-1, 1), params)
    return jnp.squeeze(out, axis=1)


# speedup vs baseline: 2.2392x; 1.2190x over previous
import functools

import jax
import jax.numpy as jnp
from jax import lax
from jax.experimental import pallas as pl
from jax.experimental.pallas import tpu as pltpu
from jax.experimental.pallas import tpu_sc as plsc

BATCH = 16384
DIM = 64
NROWS = 1000000
_FULL = (NROWS // 128) * 128
_NBLK = _FULL // 128

_NC = 2
_NS = 16
_NW = _NC * _NS
_BPW = BATCH // _NW
_CHUNK = 128


def _transpose_block(in_b, out_b, iota16):
    rolls = [(iota16 + c) % 16 for c in range(16)]

    def per_idblk(g, _):
        idv = iota16 + g * 16
        for j0 in range(0, DIM, 16):
            for c in range(16):
                jv = rolls[c] + j0
                vals = plsc.load_gather(in_b, [jv, idv])
                plsc.store_scatter(out_b, [idv, jv], vals)
        return 0

    lax.fori_loop(0, 8, per_idblk, 0)


def _sc_relayout_body(gu_t, gi_t, out_gu, out_gi,
                      ibufs, obufs, isems, osems):
    wid = lax.axis_index("s") * _NC + lax.axis_index("c")
    iota16 = lax.iota(jnp.int32, 16)
    tabs = (gu_t, gi_t)
    outs = (out_gu, out_gi)
    npert = (_NBLK + _NW - 1) // _NW

    def blkf(i):
        return jnp.minimum(wid + i * _NW, _NBLK - 1)

    def start_in(i, s):
        col = pl.multiple_of(blkf(i) * 128, 128)
        for t in range(2):
            pltpu.make_async_copy(
                tabs[t].at[:, pl.ds(col, 128)], ibufs[t][s],
                isems[t][s]).start()

    def wait_in(s):
        for t in range(2):
            pltpu.make_async_copy(
                tabs[t].at[:, pl.ds(0, 128)], ibufs[t][s],
                isems[t][s]).wait()

    def start_out(i, s):
        for t in range(2):
            pltpu.make_async_copy(
                obufs[t][s], outs[t].at[pl.ds(blkf(i) * 128, 128)],
                osems[t][s]).start()

    def wait_out(s):
        for t in range(2):
            pltpu.make_async_copy(
                outs[t].at[pl.ds(0, 128)], obufs[t][s],
                osems[t][s]).wait()

    start_in(0, 0)
    start_in(1, 1)

    def step2(i2, _):
        for s in range(2):
            i = 2 * i2 + s
            wait_in(s)

            @pl.when(i2 > 0)
            def _():
                wait_out(s)

            for t in range(2):
                _transpose_block(ibufs[t][s], obufs[t][s], iota16)
            start_in(i + 2, s)
            start_out(i, s)
        return 0

    lax.fori_loop(0, (npert + 1) // 2, step2, 0)
    wait_in(0)
    wait_in(1)
    wait_out(0)
    wait_out(1)


@functools.cache
def _make_sc_relayout():
    def body(gu_t, gi_t, out_gu, out_gi,
             ia0, ia1, ib0, ib1, oa0, oa1, ob0, ob1,
             sia0, sia1, sib0, sib1, soa0, soa1, sob0, sob1):
        _sc_relayout_body(gu_t, gi_t, out_gu, out_gi,
                          ((ia0, ia1), (ib0, ib1)),
                          ((oa0, oa1), (ob0, ob1)),
                          ((sia0, sia1), (sib0, sib1)),
                          ((soa0, soa1), (sob0, sob1)))

    return functools.partial(
        pl.kernel,
        out_type=[jax.ShapeDtypeStruct((NROWS, DIM), jnp.float32)] * 2,
        mesh=plsc.VectorSubcoreMesh(core_axis_name="c", subcore_axis_name="s"),
        compiler_params=pltpu.CompilerParams(needs_layout_passes=False),
        scratch_types=[pltpu.VMEM((DIM, 128), jnp.float32)] * 4
                     + [pltpu.VMEM((128, DIM), jnp.float32)] * 4
                     + [pltpu.SemaphoreType.DMA] * 8,
    )(body)


def _sc_gather_body(uc, ic, uids, iids, gu_t, gi_t, mu_t, mi_t,
                    out_gu, out_gi, out_mu, out_mi,
                    idx_vm, bufs, sems):
    wid = lax.axis_index("s") * _NC + lax.axis_index("c")
    base = wid * _BPW
    srcs = (uc, ic, uids, iids)
    for t in range(4):
        pltpu.sync_copy(srcs[t].at[pl.ds(base, _BPW)],
                        idx_vm.at[t])
    tables = (gu_t, gi_t, mu_t, mi_t)
    outs = (out_gu, out_gi, out_mu, out_mi)

    for c in range(_BPW // _CHUNK):
        def issue(g, _):
            vecs = [idx_vm[t, pl.ds(c * _CHUNK + g * 16, 16)]
                    for t in range(4)]
            for j in range(16):
                k = g * 16 + j
                for t in range(4):
                    pltpu.make_async_copy(
                        tables[t].at[pl.ds(vecs[t][j], 1)],
                        bufs[t].at[pl.ds(k, 1)], sems[t]).start()
            return 0

        lax.fori_loop(0, _CHUNK // 16, issue, 0)
        for t in range(4):
            pltpu.make_async_copy(
                tables[t].at[pl.ds(0, _CHUNK)], bufs[t], sems[t]).wait()
            pltpu.sync_copy(
                bufs[t], outs[t].at[pl.ds(base + c * _CHUNK, _CHUNK)])


@functools.cache
def _make_sc_gather():
    def body(uc, ic, uids, iids, gu_t, gi_t, mu_t, mi_t,
             out_gu, out_gi, out_mu, out_mi,
             idx_vm, b0, b1, b2, b3, s0, s1, s2, s3):
        _sc_gather_body(uc, ic, uids, iids, gu_t, gi_t, mu_t, mi_t,
                        out_gu, out_gi, out_mu, out_mi,
                        idx_vm, (b0, b1, b2, b3), (s0, s1, s2, s3))

    return functools.partial(
        pl.kernel,
        out_type=[jax.ShapeDtypeStruct((BATCH, DIM), jnp.float32)] * 4,
        mesh=plsc.VectorSubcoreMesh(core_axis_name="c", subcore_axis_name="s"),
        scratch_types=[
            pltpu.VMEM((4, _BPW), jnp.int32),
        ] + [pltpu.VMEM((_CHUNK, DIM), jnp.float32)] * 4
          + [pltpu.SemaphoreType.DMA] * 4,
    )(body)


_BLK = 2048


def _tc_mlp_body(gu, gi, mu, mi, uid, iid, tgu, tgi,
                 w0u, w0i, b0, s0, t0,
                 w1, b1, s1, t1,
                 w2, b2, s2, t2,
                 wg, wx, bo, out):
    iota = lax.broadcasted_iota(jnp.int32, (1, DIM), 1)
    du = uid[...] - _FULL
    di = iid[...] - _FULL
    oh_u = (du == iota).astype(jnp.float32)
    oh_i = (di == iota).astype(jnp.float32)
    gu_x = jnp.where(du >= 0, oh_u @ tgu[...], gu[...])
    gi_x = jnp.where(di >= 0, oh_i @ tgi[...], gi[...])

    x = mu[...] @ w0u[...] + mi[...] @ w0i[...] + b0[...]
    x = jnp.maximum(x, 0.0) * s0[...] + t0[...]
    x = x @ w1[...] + b1[...]
    x = jnp.maximum(x, 0.0) * s1[...] + t1[...]
    x = x @ w2[...] + b2[...]
    x = jnp.maximum(x, 0.0) * s2[...] + t2[...]
    g = gu_x * gi_x
    logit = (jnp.sum(g * wg[...], axis=1, keepdims=True)
             + jnp.sum(x * wx[...], axis=1, keepdims=True) + bo[...])
    out[...] = jax.nn.sigmoid(logit)


def _tc_mlp(gu, gi, mu, mi, uid2, iid2, params):
    n_blk = BATCH // _BLK
    data_spec = pl.BlockSpec((_BLK, DIM), lambda i: (i, 0))
    id_spec = pl.BlockSpec((_BLK, 1), lambda i: (i, 0))

    def full(a):
        return pl.BlockSpec(a.shape, lambda i: (0,) * a.ndim)

    in_specs = ([data_spec] * 4 + [id_spec] * 2
                + [full(p) for p in params])
    return pl.pallas_call(
        _tc_mlp_body,
        grid=(n_blk,),
        in_specs=in_specs,
        out_specs=pl.BlockSpec((_BLK, 1), lambda i: (i, 0)),
        out_shape=jax.ShapeDtypeStruct((BATCH, 1), jnp.float32),
    )(gu, gi, mu, mi, uid2, iid2, *params)


def kernel(inputs, gmf_user_table, gmf_item_table, mlp_user_table, mlp_item_table,
           W0, b0, g0, be0, m0, v0,
           W1, b1, g1, be1, m1, v1,
           W2, b2, g2, be2, m2, v2,
           Wout, bout):
    uids = inputs[:, 0].astype(jnp.int32)
    iids = inputs[:, 1].astype(jnp.int32)
    uc = jnp.minimum(uids, _FULL - 1)
    ic = jnp.minimum(iids, _FULL - 1)

    rgu, rgi = _make_sc_relayout()(gmf_user_table.T, gmf_item_table.T)

    gu, gi, mu, mi = _make_sc_gather()(
        uc, ic, uids, iids, rgu, rgi, mlp_user_table, mlp_item_table)

    def fold(g, be, m, v):
        s = g / jnp.sqrt(v + 1e-3)
        return s, be - m * s

    s0, t0 = fold(g0, be0, m0, v0)
    s1, t1 = fold(g1, be1, m1, v1)
    s2, t2 = fold(g2, be2, m2, v2)

    def row(a):
        return a.reshape(1, -1)

    tgu = gmf_user_table[_FULL:]
    tgi = gmf_item_table[_FULL:]

    params = [
        tgu, tgi,
        W0[:DIM], W0[DIM:], row(b0), row(s0), row(t0),
        W1, row(b1), row(s1), row(t1),
        W2, row(b2), row(s2), row(t2),
        row(Wout[:DIM, 0]), row(Wout[DIM:, 0]), row(bout),
    ]
    out = _tc_mlp(gu, gi, mu, mi,
                  uids.reshape(-1, 1), iids.reshape(-1, 1), params)
    return jnp.squeeze(out, axis=1)


# speedup vs baseline: 2.4057x; 1.0743x over previous
import functools

import jax
import jax.numpy as jnp
from jax import lax
from jax.experimental import pallas as pl
from jax.experimental.pallas import tpu as pltpu
from jax.experimental.pallas import tpu_sc as plsc

BATCH = 16384
DIM = 64

_NC = 2
_NS = 16
_NW = _NC * _NS
_BPW = BATCH // _NW
_CHUNK = 128


def _sc_gather_body(uids, iids, gu_t, gi_t, mu_t, mi_t,
                    out_gu, out_gi, out_mu, out_mi,
                    uid_vm, iid_vm, bufs, sems):
    wid = lax.axis_index("s") * _NC + lax.axis_index("c")
    base = wid * _BPW
    pltpu.sync_copy(uids.at[pl.ds(base, _BPW)], uid_vm)
    pltpu.sync_copy(iids.at[pl.ds(base, _BPW)], iid_vm)
    tables = (gu_t, gi_t, mu_t, mi_t)
    outs = (out_gu, out_gi, out_mu, out_mi)

    for c in range(_BPW // _CHUNK):
        def issue(g, _):
            uvec = uid_vm[pl.ds(c * _CHUNK + g * 16, 16)]
            ivec = iid_vm[pl.ds(c * _CHUNK + g * 16, 16)]
            for j in range(16):
                k = g * 16 + j
                rows = (uvec[j], ivec[j], uvec[j], ivec[j])
                for t in range(4):
                    pltpu.make_async_copy(
                        tables[t].at[pl.ds(rows[t], 1)],
                        bufs[t].at[pl.ds(k, 1)], sems[t]).start()
            return 0

        lax.fori_loop(0, _CHUNK // 16, issue, 0)
        for t in range(4):
            pltpu.make_async_copy(
                tables[t].at[pl.ds(0, _CHUNK)], bufs[t], sems[t]).wait()
            pltpu.sync_copy(
                bufs[t], outs[t].at[pl.ds(base + c * _CHUNK, _CHUNK)])


@functools.cache
def _make_sc_gather():
    def body(uids, iids, gu_t, gi_t, mu_t, mi_t,
             out_gu, out_gi, out_mu, out_mi,
             uid_vm, iid_vm, b0, b1, b2, b3, s0, s1, s2, s3):
        _sc_gather_body(uids, iids, gu_t, gi_t, mu_t, mi_t,
                        out_gu, out_gi, out_mu, out_mi,
                        uid_vm, iid_vm,
                        (b0, b1, b2, b3), (s0, s1, s2, s3))

    return functools.partial(
        pl.kernel,
        out_type=[jax.ShapeDtypeStruct((BATCH, DIM), jnp.float32)] * 4,
        mesh=plsc.VectorSubcoreMesh(core_axis_name="c", subcore_axis_name="s"),
        scratch_types=[
            pltpu.VMEM((_BPW,), jnp.int32),
            pltpu.VMEM((_BPW,), jnp.int32),
        ] + [pltpu.VMEM((_CHUNK, DIM), jnp.float32)] * 4
          + [pltpu.SemaphoreType.DMA] * 4,
    )(body)


_BLK = 2048


def _tc_mlp_body(gu, gi, mu, mi,
                 w0u, w0i, b0, s0, t0,
                 w1, b1, s1, t1,
                 w2, b2, s2, t2,
                 wg, wx, bo, out):
    x = mu[...] @ w0u[...] + mi[...] @ w0i[...] + b0[...]
    x = jnp.maximum(x, 0.0) * s0[...] + t0[...]
    x = x @ w1[...] + b1[...]
    x = jnp.maximum(x, 0.0) * s1[...] + t1[...]
    x = x @ w2[...] + b2[...]
    x = jnp.maximum(x, 0.0) * s2[...] + t2[...]
    g = gu[...] * gi[...]
    logit = (jnp.sum(g * wg[...], axis=1, keepdims=True)
             + jnp.sum(x * wx[...], axis=1, keepdims=True) + bo[...])
    out[...] = jax.nn.sigmoid(logit)


def _tc_mlp(gu, gi, mu, mi, params):
    n_blk = BATCH // _BLK
    data_spec = pl.BlockSpec((_BLK, DIM), lambda i: (i, 0))

    def full(a):
        return pl.BlockSpec(a.shape, lambda i: (0,) * a.ndim)

    in_specs = [data_spec] * 4 + [full(p) for p in params]
    return pl.pallas_call(
        _tc_mlp_body,
        grid=(n_blk,),
        in_specs=in_specs,
        out_specs=pl.BlockSpec((_BLK, 1), lambda i: (i, 0)),
        out_shape=jax.ShapeDtypeStruct((BATCH, 1), jnp.float32),
    )(gu, gi, mu, mi, *params)


def kernel(inputs, gmf_user_table, gmf_item_table, mlp_user_table, mlp_item_table,
           W0, b0, g0, be0, m0, v0,
           W1, b1, g1, be1, m1, v1,
           W2, b2, g2, be2, m2, v2,
           Wout, bout):
    uids = inputs[:, 0].astype(jnp.int32)
    iids = inputs[:, 1].astype(jnp.int32)

    gu, gi, mu, mi = _make_sc_gather()(
        uids, iids, gmf_user_table, gmf_item_table,
        mlp_user_table, mlp_item_table)

    def fold(g, be, m, v):
        s = g / jnp.sqrt(v + 1e-3)
        return s, be - m * s

    s0, t0 = fold(g0, be0, m0, v0)
    s1, t1 = fold(g1, be1, m1, v1)
    s2, t2 = fold(g2, be2, m2, v2)

    def row(a):
        return a.reshape(1, -1)

    params = [
        W0[:DIM], W0[DIM:], row(b0), row(s0), row(t0),
        W1, row(b1), row(s1), row(t1),
        W2, row(b2), row(s2), row(t2),
        row(Wout[:DIM, 0]), row(Wout[DIM:, 0]), row(bout),
    ]
    out = _tc_mlp(gu, gi, mu, mi, params)
    return jnp.squeeze(out, axis=1)
